# baseline (speedup 1.0000x reference)
import dataclasses
import functools

import jax
import jax.numpy as jnp
from jax import lax
from jax.experimental import pallas as pl
from jax.experimental.pallas import tpu as pltpu
from jax.experimental.pallas import tpu_sc as plsc

B = 16384
D = 64
NC = 2
NS = 16
L = 16
NW = NC * NS
BPW = B // NW
CHUNK = 128
NCHUNK = BPW // CHUNK
NG = BPW // L
DU = 8


def _glove_body(ci_hbm, ti_hbm, co_hbm, we_hbm, ev_hbm, eu_hbm, vb_hbm,
                ub_hbm, out_hbm,
                ci_v, ti_v, cr_v, tr_v, cb_v, tb_v, co_v, we_v,
                red_shared, red_v, tot_v, sem):
    cid = lax.axis_index("c")
    sid = lax.axis_index("s")
    wid = cid * NS + sid

    pltpu.sync_copy(ci_hbm.at[wid], ci_v)
    pltpu.sync_copy(ti_hbm.at[wid], ti_v)

    copies = []
    for j in range(NCHUNK):
        rs = pl.ds(j * CHUNK, CHUNK)
        copies.append(pltpu.async_copy(ev_hbm.at[ci_v.at[j]], cr_v.at[rs], sem))
        copies.append(pltpu.async_copy(vb_hbm.at[ci_v.at[j]], cb_v.at[rs], sem))
        copies.append(pltpu.async_copy(eu_hbm.at[ti_v.at[j]], tr_v.at[rs], sem))
        copies.append(pltpu.async_copy(ub_hbm.at[ti_v.at[j]], tb_v.at[rs], sem))
    pltpu.sync_copy(co_hbm.at[wid], co_v)
    pltpu.sync_copy(we_hbm.at[wid], we_v)
    for cp in copies:
        cp.wait()

    lane = lax.iota(jnp.int32, L)

    def group_body(g, acc):
        row0 = g * L
        rows = row0 + lane

        def d_body(dd, ip):
            d0 = dd * DU
            for k in range(DU):
                c = jnp.full((L,), d0 + k, dtype=jnp.int32)
                cv = plsc.load_gather(cr_v, [rows, c])
                tv = plsc.load_gather(tr_v, [rows, c])
                ip = ip + cv * tv
            return ip

        ip = lax.fori_loop(0, D // DU, d_body, jnp.zeros((L,), jnp.float32))
        cb = cb_v[pl.ds(row0, L)]
        tb = tb_v[pl.ds(row0, L)]
        co = co_v[pl.ds(row0, L)]
        we = we_v[pl.ds(row0, L)]
        e = ip + cb + tb - co
        return acc + we * e * e

    part = lax.fori_loop(0, NG, group_body, jnp.zeros((L,), jnp.float32))

    tot_v[...] = part
    pltpu.sync_copy(tot_v, red_shared.at[sid])
    plsc.subcore_barrier()

    @pl.when(sid == 0)
    def _():
        pltpu.sync_copy(red_shared, red_v)
        s = jnp.zeros((L,), jnp.float32)
        for i in range(NS):
            s = s + red_v[i]
        total = jnp.sum(s)
        tot_v[...] = jnp.full((L,), total, dtype=jnp.float32)
        pltpu.sync_copy(tot_v, out_hbm.at[cid])


_cp = pltpu.CompilerParams()
if "needs_layout_passes" in pltpu.CompilerParams.__dataclass_fields__:
    _cp = dataclasses.replace(_cp, needs_layout_passes=False)
if "use_tc_tiling_on_sc" in pltpu.CompilerParams.__dataclass_fields__:
    _cp = dataclasses.replace(_cp, use_tc_tiling_on_sc=False)

_glove_call = functools.partial(
    pl.kernel,
    compiler_params=_cp,
    out_type=jax.ShapeDtypeStruct((NC, L), jnp.float32),
    mesh=plsc.VectorSubcoreMesh(core_axis_name="c", subcore_axis_name="s"),
    scratch_types=[
        pltpu.VMEM((NCHUNK, CHUNK), jnp.int32),
        pltpu.VMEM((NCHUNK, CHUNK), jnp.int32),
        pltpu.VMEM((BPW, D), jnp.float32),
        pltpu.VMEM((BPW, D), jnp.float32),
        pltpu.VMEM((BPW,), jnp.float32),
        pltpu.VMEM((BPW,), jnp.float32),
        pltpu.VMEM((BPW,), jnp.float32),
        pltpu.VMEM((BPW,), jnp.float32),
        pltpu.VMEM_SHARED((NS, L), jnp.float32),
        pltpu.VMEM((NS, L), jnp.float32),
        pltpu.VMEM((L,), jnp.float32),
        pltpu.SemaphoreType.DMA,
    ],
)(_glove_body)


def kernel(center_words, target_words, coocs, weighting, emb_v, emb_u,
           v_bias, u_bias):
    ci = center_words.reshape(NW, NCHUNK, CHUNK).astype(jnp.int32)
    ti = target_words.reshape(NW, NCHUNK, CHUNK).astype(jnp.int32)
    co = coocs.reshape(NW, BPW)
    we = weighting.reshape(NW, BPW)
    vb = v_bias.reshape(-1)
    ub = u_bias.reshape(-1)
    out = _glove_call(ci, ti, co, we, emb_v, emb_u, vb, ub)
    return out[0, 0] + out[1, 0]
